# baseline (speedup 1.0000x reference)
import jax
import jax.numpy as jnp
from jax import lax
from jax.experimental import pallas as pl
from jax.experimental.pallas import tpu as pltpu
from jax.experimental.pallas import tpu_sc as plsc

BATCH = 16384
DIM = 128
NW = 32
PER_W = BATCH // NW
CHUNK = 128
NCHUNK = PER_W // CHUNK
LANES = 16


def _fast_sqrt(x):
    i = lax.bitcast_convert_type(x, jnp.int32)
    i = 0x5F3759DF - lax.shift_right_logical(i, 1)
    y = lax.bitcast_convert_type(i, jnp.float32)
    xhalf = 0.5 * x
    for _ in range(3):
        y = y * (1.5 - xhalf * y * y)
    return x * y


def _sc_body(h_hbm, r_hbm, t_hbm, ent_hbm, rel_hbm, out_hbm,
             hidx, ridx, tidx, hrows, rrows, trows, out_v, sq_v, sem0, sem1):
    wid = lax.axis_index("s") * 2 + lax.axis_index("c")
    base = wid * PER_W
    lane = lax.iota(jnp.int32, LANES)
    sems = (sem0, sem1)

    def issue(j):
        b = j % 2
        off = base + j * CHUNK
        pltpu.sync_copy(h_hbm.at[pl.ds(off, CHUNK)], hidx.at[b])
        pltpu.sync_copy(r_hbm.at[pl.ds(off, CHUNK)], ridx.at[b])
        pltpu.sync_copy(t_hbm.at[pl.ds(off, CHUNK)], tidx.at[b])
        return (
            pltpu.async_copy(ent_hbm.at[hidx.at[b]], hrows.at[b], sems[b]),
            pltpu.async_copy(rel_hbm.at[ridx.at[b]], rrows.at[b], sems[b]),
            pltpu.async_copy(ent_hbm.at[tidx.at[b]], trows.at[b], sems[b]),
        )

    inflight = {0: issue(0)}
    for j in range(NCHUNK):
        b = j % 2
        if j + 1 < NCHUNK:
            inflight[j + 1] = issue(j + 1)
        for cp in inflight.pop(j):
            cp.wait()

        @plsc.parallel_loop(0, CHUNK, step=4, unroll=1)
        def row4_body(r0, b=b):
            for k in range(4):
                row = r0 + k
                acc0 = jnp.zeros((LANES,), jnp.float32)
                acc1 = jnp.zeros((LANES,), jnp.float32)
                for c in range(DIM // LANES):
                    sl = pl.ds(c * LANES, LANES)
                    d = hrows[b, row, sl] + rrows[b, row, sl] - trows[b, row, sl]
                    if c % 2:
                        acc1 = acc1 + d * d
                    else:
                        acc0 = acc0 + d * d
                sq_v[row, :] = acc0 + acc1

        def merge_body(g, _, j=j):
            vecs = [sq_v[g * LANES + jj, :] for jj in range(LANES)]
            for s in (1, 2, 4, 8):
                nb = (lane & s) == 0
                nxt = []
                for i2 in range(0, len(vecs), 2):
                    u, v = vecs[i2], vecs[i2 + 1]
                    w = jnp.where(nb, u, v) + jnp.where(nb, v, u).at[
                        lane ^ s].get(mode="promise_in_bounds",
                                      unique_indices=True)
                    nxt.append(w)
                vecs = nxt
            out_v[pl.ds(j * CHUNK + g * LANES, LANES)] = _fast_sqrt(vecs[0])
            return 0

        lax.fori_loop(0, CHUNK // LANES, merge_body, 0)

    pltpu.sync_copy(out_v, out_hbm.at[pl.ds(base, PER_W)])


@jax.jit
def kernel(h, r, t, entity_emb, relation_emb):
    mesh = plsc.VectorSubcoreMesh(core_axis_name="c", subcore_axis_name="s")
    run = pl.kernel(
        _sc_body,
        out_type=jax.ShapeDtypeStruct((BATCH,), jnp.float32),
        mesh=mesh,
        scratch_types=[
            pltpu.VMEM((2, CHUNK), jnp.int32),
            pltpu.VMEM((2, CHUNK), jnp.int32),
            pltpu.VMEM((2, CHUNK), jnp.int32),
            pltpu.VMEM((2, CHUNK, DIM), jnp.float32),
            pltpu.VMEM((2, CHUNK, DIM), jnp.float32),
            pltpu.VMEM((2, CHUNK, DIM), jnp.float32),
            pltpu.VMEM((PER_W,), jnp.float32),
            pltpu.VMEM((CHUNK, LANES), jnp.float32),
            pltpu.SemaphoreType.DMA,
            pltpu.SemaphoreType.DMA,
        ],
    )
    return run(h.astype(jnp.int32), r.astype(jnp.int32), t.astype(jnp.int32),
               entity_emb, relation_emb)


# speedup vs baseline: 1.0594x; 1.0594x over previous
import jax
import jax.numpy as jnp
from jax import lax
from jax.experimental import pallas as pl
from jax.experimental.pallas import tpu as pltpu
from jax.experimental.pallas import tpu_sc as plsc

BATCH = 16384
DIM = 128
NW = 32
PER_W = BATCH // NW
CHUNK = 128
NCHUNK = PER_W // CHUNK
LANES = 16


def _fast_sqrt(x):
    i = lax.bitcast_convert_type(x, jnp.int32)
    i = 0x5F3759DF - lax.shift_right_logical(i, 1)
    y = lax.bitcast_convert_type(i, jnp.float32)
    xhalf = 0.5 * x
    for _ in range(3):
        y = y * (1.5 - xhalf * y * y)
    return x * y


def _sc_body(h_hbm, r_hbm, t_hbm, ent_hbm, rel_hbm, out_hbm,
             hidx, ridx, tidx, hrows, rrows, trows, out_v, sq_v,
             sem0, sem1, sem_idx):
    wid = lax.axis_index("s") * 2 + lax.axis_index("c")
    base = wid * PER_W
    lane = lax.iota(jnp.int32, LANES)
    sems = (sem0, sem1)

    idx_cps = [
        pltpu.async_copy(src.at[pl.ds(base, PER_W)], dst, sem_idx)
        for src, dst in ((h_hbm, hidx), (r_hbm, ridx), (t_hbm, tidx))
    ]
    for cp in idx_cps:
        cp.wait()

    def issue(j):
        b = j % 2
        sl = pl.ds(j * CHUNK, CHUNK)
        return (
            pltpu.async_copy(ent_hbm.at[hidx.at[sl]], hrows.at[b], sems[b]),
            pltpu.async_copy(rel_hbm.at[ridx.at[sl]], rrows.at[b], sems[b]),
            pltpu.async_copy(ent_hbm.at[tidx.at[sl]], trows.at[b], sems[b]),
        )

    inflight = {0: issue(0)}
    for j in range(NCHUNK):
        b = j % 2
        if j + 1 < NCHUNK:
            inflight[j + 1] = issue(j + 1)
        for cp in inflight.pop(j):
            cp.wait()

        def row4_body(i, _, b=b):
            r0 = i * 4
            for k in range(4):
                row = r0 + k
                acc0 = jnp.zeros((LANES,), jnp.float32)
                acc1 = jnp.zeros((LANES,), jnp.float32)
                for c in range(DIM // LANES):
                    sl = pl.ds(c * LANES, LANES)
                    d = hrows[b, row, sl] + rrows[b, row, sl] - trows[b, row, sl]
                    if c % 2:
                        acc1 = acc1 + d * d
                    else:
                        acc0 = acc0 + d * d
                sq_v[row, :] = acc0 + acc1
            return 0

        lax.fori_loop(0, CHUNK // 4, row4_body, 0)

        def merge_body(g, _, j=j):
            vecs = [sq_v[g * LANES + jj, :] for jj in range(LANES)]
            for s in (1, 2, 4, 8):
                nb = (lane & s) == 0
                nxt = []
                for i2 in range(0, len(vecs), 2):
                    u, v = vecs[i2], vecs[i2 + 1]
                    w = jnp.where(nb, u, v) + jnp.where(nb, v, u).at[
                        lane ^ s].get(mode="promise_in_bounds",
                                      unique_indices=True)
                    nxt.append(w)
                vecs = nxt
            out_v[pl.ds(j * CHUNK + g * LANES, LANES)] = _fast_sqrt(vecs[0])
            return 0

        lax.fori_loop(0, CHUNK // LANES, merge_body, 0)

    pltpu.sync_copy(out_v, out_hbm.at[pl.ds(base, PER_W)])


@jax.jit
def kernel(h, r, t, entity_emb, relation_emb):
    mesh = plsc.VectorSubcoreMesh(core_axis_name="c", subcore_axis_name="s")
    run = pl.kernel(
        _sc_body,
        out_type=jax.ShapeDtypeStruct((BATCH,), jnp.float32),
        mesh=mesh,
        scratch_types=[
            pltpu.VMEM((PER_W,), jnp.int32),
            pltpu.VMEM((PER_W,), jnp.int32),
            pltpu.VMEM((PER_W,), jnp.int32),
            pltpu.VMEM((2, CHUNK, DIM), jnp.float32),
            pltpu.VMEM((2, CHUNK, DIM), jnp.float32),
            pltpu.VMEM((2, CHUNK, DIM), jnp.float32),
            pltpu.VMEM((PER_W,), jnp.float32),
            pltpu.VMEM((CHUNK, LANES), jnp.float32),
            pltpu.SemaphoreType.DMA,
            pltpu.SemaphoreType.DMA,
            pltpu.SemaphoreType.DMA,
        ],
    )
    return run(h.astype(jnp.int32), r.astype(jnp.int32), t.astype(jnp.int32),
               entity_emb, relation_emb)
